# Initial kernel scaffold; baseline (speedup 1.0000x reference)
import jax
import jax.numpy as jnp
from jax.experimental import pallas as pl


def kernel(x, edge_index, W1, b1, W2, b2):
    raise NotImplementedError("write your pallas kernel here")


# speedup vs baseline: 9.0440x; 9.0440x over previous
import functools

import jax
import jax.numpy as jnp
from jax import lax
from jax.experimental import pallas as pl
from jax.experimental.pallas import tpu as pltpu
from jax.experimental.pallas import tpu_sc as plsc

N = 10000
E = 320000
D = 128
NP = 10240
NW = 32
NS = 16
CHUNK = 128
NCHUNK = 80
EPW = CHUNK * NCHUNK
EPAD = EPW * NW
RPT = NP // NS

_MESH = plsc.VectorSubcoreMesh(core_axis_name="c", subcore_axis_name="s")


@functools.partial(
    pl.kernel,
    mesh=_MESH,
    out_type=jax.ShapeDtypeStruct((2, NP), jnp.float32),
    scratch_types=[
        pltpu.VMEM((NCHUNK, CHUNK), jnp.int32),
        pltpu.VMEM((CHUNK,), jnp.float32),
        pltpu.VMEM_SHARED((NP,), jnp.float32),
    ],
)
def _sc_degree(dst_hbm, ones_hbm, zcol_hbm, out_hbm, dst_v, ones_v, deg_sh):
    c = lax.axis_index("c")
    s = lax.axis_index("s")
    wid = s * 2 + c
    pltpu.sync_copy(zcol_hbm, deg_sh.at[pl.ds(s * RPT, RPT)])
    pltpu.sync_copy(dst_hbm.at[wid], dst_v)
    pltpu.sync_copy(ones_hbm, ones_v)
    plsc.subcore_barrier()

    def chunk(j, carry):
        pltpu.sync_copy(ones_v, deg_sh.at[dst_v.at[j]], add=True)
        return carry

    lax.fori_loop(0, NCHUNK, chunk, 0)
    plsc.subcore_barrier()
    pltpu.sync_copy(deg_sh.at[pl.ds(s * RPT, RPT)],
                    out_hbm.at[c, pl.ds(s * RPT, RPT)])


@functools.partial(
    pl.kernel,
    mesh=_MESH,
    out_type=jax.ShapeDtypeStruct((2, NP, D), jnp.float32),
    scratch_types=[
        pltpu.VMEM((NCHUNK, CHUNK), jnp.int32),
        pltpu.VMEM((NCHUNK, CHUNK), jnp.int32),
        pltpu.VMEM((CHUNK, D), jnp.float32),
        pltpu.VMEM_SHARED((NP, D), jnp.float32),
    ],
)
def _sc_scatter(y_hbm, src_hbm, dst_hbm, zrows_hbm, out_hbm,
                src_v, dst_v, rows_v, acc_sh):
    c = lax.axis_index("c")
    s = lax.axis_index("s")
    wid = s * 2 + c
    pltpu.sync_copy(zrows_hbm, acc_sh.at[pl.ds(s * RPT, RPT)])
    pltpu.sync_copy(src_hbm.at[wid], src_v)
    pltpu.sync_copy(dst_hbm.at[wid], dst_v)
    plsc.subcore_barrier()

    def chunk(j, carry):
        pltpu.sync_copy(y_hbm.at[src_v.at[j]], rows_v)
        pltpu.sync_copy(rows_v, acc_sh.at[dst_v.at[j]], add=True)
        return carry

    lax.fori_loop(0, NCHUNK, chunk, 0)
    plsc.subcore_barrier()
    pltpu.sync_copy(acc_sh.at[pl.ds(s * RPT, RPT)],
                    out_hbm.at[c, pl.ds(s * RPT, RPT)])


_BLK = 1024
_GRID = NP // _BLK


def _t1_body(x_ref, w_ref, deg_ref, y_ref, dinv_ref):
    i = pl.program_id(0)
    xw = jnp.dot(x_ref[...], w_ref[...], preferred_element_type=jnp.float32)
    deg = deg_ref[0] + deg_ref[1] + 1.0
    rows = lax.broadcasted_iota(jnp.int32, (_BLK, 1), 0) + i * _BLK
    dinv = jnp.where(rows < N, lax.rsqrt(deg), 0.0)
    y_ref[...] = xw * dinv
    dinv_ref[...] = dinv


def _t2_body(a_ref, y_ref, d_ref, b_ref, w_ref, o_ref):
    d = d_ref[...]
    h = jnp.maximum(d * (a_ref[0] + a_ref[1] + y_ref[...]) + b_ref[...], 0.0)
    o_ref[...] = jnp.dot(h, w_ref[...], preferred_element_type=jnp.float32) * d


def _t3_body(a_ref, y_ref, d_ref, b_ref, o_ref):
    o_ref[...] = (d_ref[...] * (a_ref[0] + a_ref[1] + y_ref[...])
                  + b_ref[...])


_acc_spec = pl.BlockSpec((2, _BLK, D), lambda i: (0, i, 0))
_row_spec = pl.BlockSpec((_BLK, D), lambda i: (i, 0))
_col_spec = pl.BlockSpec((_BLK, 1), lambda i: (i, 0))
_w_spec = pl.BlockSpec((D, D), lambda i: (0, 0))
_b_spec = pl.BlockSpec((1, D), lambda i: (0, 0))
_deg_spec = pl.BlockSpec((2, _BLK, 1), lambda i: (0, i, 0))

_t1 = pl.pallas_call(
    _t1_body,
    grid=(_GRID,),
    in_specs=[_row_spec, _w_spec, _deg_spec],
    out_specs=[_row_spec, _col_spec],
    out_shape=[jax.ShapeDtypeStruct((NP, D), jnp.float32),
               jax.ShapeDtypeStruct((NP, 1), jnp.float32)],
)

_t2 = pl.pallas_call(
    _t2_body,
    grid=(_GRID,),
    in_specs=[_acc_spec, _row_spec, _col_spec, _b_spec, _w_spec],
    out_specs=_row_spec,
    out_shape=jax.ShapeDtypeStruct((NP, D), jnp.float32),
)

_t3 = pl.pallas_call(
    _t3_body,
    grid=(_GRID,),
    in_specs=[_acc_spec, _row_spec, _col_spec, _b_spec],
    out_specs=_row_spec,
    out_shape=jax.ShapeDtypeStruct((NP, D), jnp.float32),
)


def kernel(x, edge_index, W1, b1, W2, b2):
    src = edge_index[0].astype(jnp.int32)
    dst = edge_index[1].astype(jnp.int32)
    pad = jnp.full((EPAD - E,), NP - 1, jnp.int32)
    src_r = jnp.concatenate([src, pad]).reshape(NW, NCHUNK, CHUNK)
    dst_r = jnp.concatenate([dst, pad]).reshape(NW, NCHUNK, CHUNK)
    x_pad = jnp.pad(x, ((0, NP - N), (0, 0)))
    ones_col = jnp.ones((CHUNK,), jnp.float32)
    zeros_col = jnp.zeros((RPT,), jnp.float32)
    zeros_rows = jnp.zeros((RPT, D), jnp.float32)
    b1r = b1.reshape(1, D)
    b2r = b2.reshape(1, D)

    deg = _sc_degree(dst_r, ones_col, zeros_col).reshape(2, NP, 1)
    y1, dinv = _t1(x_pad, W1, deg)
    acc1 = _sc_scatter(y1, src_r, dst_r, zeros_rows)
    y2 = _t2(acc1, y1, dinv, b1r, W2)
    acc2 = _sc_scatter(y2, src_r, dst_r, zeros_rows)
    out = _t3(acc2, y2, dinv, b2r)
    return out[:N]
